# Initial kernel scaffold; baseline (speedup 1.0000x reference)
import jax
import jax.numpy as jnp
from jax.experimental import pallas as pl


def kernel(ids, min_embedding, delta_embedding, rel_trans_for_head, rel_scale_for_head, rel_trans_for_tail, rel_scale_for_tail):
    raise NotImplementedError("write your pallas kernel here")


# speedup vs baseline: 1.0534x; 1.0534x over previous
import functools

import jax
import jax.numpy as jnp
from jax import lax
from jax.experimental import pallas as pl

GUMBEL_BETA = 0.01
EULER_GAMMA = 0.5772156649015329
EMB = 128
TBL = 1024
BM = 1024


def _log1pexp(x):
    return jnp.log1p(jnp.exp(x))


def _logaddexp(a, b):
    m = jnp.maximum(a, b)
    return m + _log1pexp(-jnp.abs(a - b))


def _softplus(x):
    return jnp.maximum(x, 0.0) + _log1pexp(-jnp.abs(x))


def _log_volume(delta):
    eps = jnp.finfo(jnp.float32).tiny
    sp = _softplus(delta - 2.0 * EULER_GAMMA * GUMBEL_BETA)
    return jnp.sum(jnp.log(jnp.maximum(sp, eps)), axis=-1, keepdims=True)


def _body(h_ref, t_ref, r_ref, ent_ref, rel_ref, out_ref):
    f32 = jnp.float32
    iota = lax.broadcasted_iota(jnp.int32, (BM, TBL), 1)
    oh_h = (iota == h_ref[...]).astype(f32)
    oh_t = (iota == t_ref[...]).astype(f32)
    oh_r = (iota == r_ref[...]).astype(f32)
    hi = lax.Precision.HIGHEST
    gh = lax.dot(oh_h, ent_ref[...], precision=hi)
    gt = lax.dot(oh_t, ent_ref[...], precision=hi)
    gr = lax.dot(oh_r, rel_ref[...], precision=hi)

    min_h = gh[:, :EMB]
    max_h = min_h + jnp.exp(gh[:, EMB:])
    delta_h = max_h - min_h
    trans_h = gr[:, 0:EMB]
    scale_h = jnp.maximum(gr[:, EMB:2 * EMB], 0.0)
    min_h = min_h + trans_h
    delta_h = delta_h * scale_h
    max_h = min_h + delta_h

    min_t = gt[:, :EMB]
    max_t = min_t + jnp.exp(gt[:, EMB:])
    delta_t = max_t - min_t
    trans_t = gr[:, 2 * EMB:3 * EMB]
    scale_t = jnp.maximum(gr[:, 3 * EMB:], 0.0)
    min_t = min_t + trans_t
    delta_t = delta_t * scale_t
    max_t = min_t + delta_t

    b = GUMBEL_BETA
    int_min = b * _logaddexp(min_h / b, min_t / b)
    int_min = jnp.maximum(int_min, jnp.maximum(min_h, min_t))
    int_max = -b * _logaddexp(-max_h / b, -max_t / b)
    int_max = jnp.minimum(int_max, jnp.minimum(max_h, max_t))

    li = _log_volume(int_max - int_min)
    lt = _log_volume(delta_t)
    out_ref[...] = jnp.exp(li - lt)


def kernel(ids, min_embedding, delta_embedding, rel_trans_for_head,
           rel_scale_for_head, rel_trans_for_tail, rel_scale_for_tail):
    batch = ids.shape[0]
    h = ids[:, 0:1]
    r = ids[:, 1:2]
    t = ids[:, 2:3]
    ent = jnp.concatenate([min_embedding[:TBL], delta_embedding[:TBL]], axis=1)
    pad = TBL - rel_trans_for_head.shape[0]
    rel = jnp.concatenate([
        jnp.pad(rel_trans_for_head, ((0, pad), (0, 0))),
        jnp.pad(rel_scale_for_head, ((0, pad), (0, 0))),
        jnp.pad(rel_trans_for_tail, ((0, pad), (0, 0))),
        jnp.pad(rel_scale_for_tail, ((0, pad), (0, 0))),
    ], axis=1)

    grid = batch // BM
    out = pl.pallas_call(
        _body,
        grid=(grid,),
        in_specs=[
            pl.BlockSpec((BM, 1), lambda i: (i, 0)),
            pl.BlockSpec((BM, 1), lambda i: (i, 0)),
            pl.BlockSpec((BM, 1), lambda i: (i, 0)),
            pl.BlockSpec((TBL, 2 * EMB), lambda i: (0, 0)),
            pl.BlockSpec((TBL, 4 * EMB), lambda i: (0, 0)),
        ],
        out_specs=pl.BlockSpec((BM, 1), lambda i: (i, 0)),
        out_shape=jax.ShapeDtypeStruct((batch, 1), jnp.float32),
    )(h, t, r, ent, rel)
    return out[:, 0]


# speedup vs baseline: 2.2548x; 2.1404x over previous
import functools

import jax
import jax.numpy as jnp
from jax import lax
from jax.experimental import pallas as pl

GUMBEL_BETA = 0.01
EULER_GAMMA = 0.5772156649015329
EMB = 128
TBL = 1024
BM = 1024


def _log1pexp(x):
    return jnp.log1p(jnp.exp(x))


def _logaddexp(a, b):
    m = jnp.maximum(a, b)
    return m + _log1pexp(-jnp.abs(a - b))


def _softplus(x):
    return jnp.maximum(x, 0.0) + _log1pexp(-jnp.abs(x))


def _log_volume(delta):
    eps = jnp.finfo(jnp.float32).tiny
    sp = _softplus(delta - 2.0 * EULER_GAMMA * GUMBEL_BETA)
    return jnp.sum(jnp.log(jnp.maximum(sp, eps)), axis=-1, keepdims=True)


def _gather_mm(oh, hi_ref, lo_ref):
    f32 = jnp.float32
    return (lax.dot(oh, hi_ref[...], preferred_element_type=f32)
            + lax.dot(oh, lo_ref[...], preferred_element_type=f32))


def _body(h_ref, t_ref, r_ref, ent_hi_ref, ent_lo_ref, rel_hi_ref,
          rel_lo_ref, out_ref):
    bf16 = jnp.bfloat16
    iota = lax.broadcasted_iota(jnp.int32, (BM, TBL), 1)
    oh_h = (iota == h_ref[...]).astype(bf16)
    oh_t = (iota == t_ref[...]).astype(bf16)
    oh_r = (iota == r_ref[...]).astype(bf16)
    gh = _gather_mm(oh_h, ent_hi_ref, ent_lo_ref)
    gt = _gather_mm(oh_t, ent_hi_ref, ent_lo_ref)
    gr = _gather_mm(oh_r, rel_hi_ref, rel_lo_ref)

    min_h = gh[:, :EMB]
    max_h = min_h + jnp.exp(gh[:, EMB:])
    delta_h = max_h - min_h
    trans_h = gr[:, 0:EMB]
    scale_h = jnp.maximum(gr[:, EMB:2 * EMB], 0.0)
    min_h = min_h + trans_h
    delta_h = delta_h * scale_h
    max_h = min_h + delta_h

    min_t = gt[:, :EMB]
    max_t = min_t + jnp.exp(gt[:, EMB:])
    delta_t = max_t - min_t
    trans_t = gr[:, 2 * EMB:3 * EMB]
    scale_t = jnp.maximum(gr[:, 3 * EMB:], 0.0)
    min_t = min_t + trans_t
    delta_t = delta_t * scale_t
    max_t = min_t + delta_t

    b = GUMBEL_BETA
    int_min = b * _logaddexp(min_h / b, min_t / b)
    int_min = jnp.maximum(int_min, jnp.maximum(min_h, min_t))
    int_max = -b * _logaddexp(-max_h / b, -max_t / b)
    int_max = jnp.minimum(int_max, jnp.minimum(max_h, max_t))

    li = _log_volume(int_max - int_min)
    lt = _log_volume(delta_t)
    out_ref[...] = jnp.exp(li - lt)


def kernel(ids, min_embedding, delta_embedding, rel_trans_for_head,
           rel_scale_for_head, rel_trans_for_tail, rel_scale_for_tail):
    batch = ids.shape[0]
    h = ids[:, 0:1]
    r = ids[:, 1:2]
    t = ids[:, 2:3]
    ent = jnp.concatenate([min_embedding[:TBL], delta_embedding[:TBL]], axis=1)
    pad = TBL - rel_trans_for_head.shape[0]
    rel = jnp.concatenate([
        jnp.pad(rel_trans_for_head, ((0, pad), (0, 0))),
        jnp.pad(rel_scale_for_head, ((0, pad), (0, 0))),
        jnp.pad(rel_trans_for_tail, ((0, pad), (0, 0))),
        jnp.pad(rel_scale_for_tail, ((0, pad), (0, 0))),
    ], axis=1)
    ent_hi = ent.astype(jnp.bfloat16)
    ent_lo = (ent - ent_hi.astype(jnp.float32)).astype(jnp.bfloat16)
    rel_hi = rel.astype(jnp.bfloat16)
    rel_lo = (rel - rel_hi.astype(jnp.float32)).astype(jnp.bfloat16)

    grid = batch // BM
    out = pl.pallas_call(
        _body,
        grid=(grid,),
        in_specs=[
            pl.BlockSpec((BM, 1), lambda i: (i, 0)),
            pl.BlockSpec((BM, 1), lambda i: (i, 0)),
            pl.BlockSpec((BM, 1), lambda i: (i, 0)),
            pl.BlockSpec((TBL, 2 * EMB), lambda i: (0, 0)),
            pl.BlockSpec((TBL, 2 * EMB), lambda i: (0, 0)),
            pl.BlockSpec((TBL, 4 * EMB), lambda i: (0, 0)),
            pl.BlockSpec((TBL, 4 * EMB), lambda i: (0, 0)),
        ],
        out_specs=pl.BlockSpec((BM, 1), lambda i: (i, 0)),
        out_shape=jax.ShapeDtypeStruct((batch, 1), jnp.float32),
    )(h, t, r, ent_hi, ent_lo, rel_hi, rel_lo)
    return out[:, 0]


# speedup vs baseline: 2.2556x; 1.0004x over previous
import functools

import jax
import jax.numpy as jnp
from jax import lax
from jax.experimental import pallas as pl

GUMBEL_BETA = 0.01
EULER_GAMMA = 0.5772156649015329
EMB = 128
TBL = 1024
BM = 1024


def _log1pexp(x):
    return jnp.log1p(jnp.exp(x))


def _logaddexp(a, b):
    m = jnp.maximum(a, b)
    return m + _log1pexp(-jnp.abs(a - b))


def _softplus(x):
    return jnp.maximum(x, 0.0) + _log1pexp(-jnp.abs(x))


def _log_volume(delta):
    eps = jnp.finfo(jnp.float32).tiny
    sp = _softplus(delta - 2.0 * EULER_GAMMA * GUMBEL_BETA)
    return jnp.sum(jnp.log(jnp.maximum(sp, eps)), axis=-1, keepdims=True)


def _body(h_ref, t_ref, r_ref, ent_ref, rel_ref, out_ref):
    bf16 = jnp.bfloat16
    f32 = jnp.float32
    iota = lax.broadcasted_iota(jnp.int32, (BM, 2 * TBL), 1) & (TBL - 1)
    oh_h = (iota == h_ref[...]).astype(bf16)
    oh_t = (iota == t_ref[...]).astype(bf16)
    oh_r = (iota == r_ref[...]).astype(bf16)
    gh = lax.dot(oh_h, ent_ref[...], preferred_element_type=f32)
    gt = lax.dot(oh_t, ent_ref[...], preferred_element_type=f32)
    gr = lax.dot(oh_r, rel_ref[...], preferred_element_type=f32)

    min_h = gh[:, :EMB]
    max_h = min_h + jnp.exp(gh[:, EMB:])
    delta_h = max_h - min_h
    trans_h = gr[:, 0:EMB]
    scale_h = jnp.maximum(gr[:, EMB:2 * EMB], 0.0)
    min_h = min_h + trans_h
    delta_h = delta_h * scale_h
    max_h = min_h + delta_h

    min_t = gt[:, :EMB]
    max_t = min_t + jnp.exp(gt[:, EMB:])
    delta_t = max_t - min_t
    trans_t = gr[:, 2 * EMB:3 * EMB]
    scale_t = jnp.maximum(gr[:, 3 * EMB:], 0.0)
    min_t = min_t + trans_t
    delta_t = delta_t * scale_t
    max_t = min_t + delta_t

    b = GUMBEL_BETA
    int_min = b * _logaddexp(min_h / b, min_t / b)
    int_min = jnp.maximum(int_min, jnp.maximum(min_h, min_t))
    int_max = -b * _logaddexp(-max_h / b, -max_t / b)
    int_max = jnp.minimum(int_max, jnp.minimum(max_h, max_t))

    li = _log_volume(int_max - int_min)
    lt = _log_volume(delta_t)
    out_ref[...] = jnp.exp(li - lt)


def kernel(ids, min_embedding, delta_embedding, rel_trans_for_head,
           rel_scale_for_head, rel_trans_for_tail, rel_scale_for_tail):
    batch = ids.shape[0]
    h = ids[:, 0:1]
    r = ids[:, 1:2]
    t = ids[:, 2:3]
    ent = jnp.concatenate([min_embedding[:TBL], delta_embedding[:TBL]], axis=1)
    pad = TBL - rel_trans_for_head.shape[0]
    rel = jnp.concatenate([
        jnp.pad(rel_trans_for_head, ((0, pad), (0, 0))),
        jnp.pad(rel_scale_for_head, ((0, pad), (0, 0))),
        jnp.pad(rel_trans_for_tail, ((0, pad), (0, 0))),
        jnp.pad(rel_scale_for_tail, ((0, pad), (0, 0))),
    ], axis=1)
    def _hilo(x):
        hi = x.astype(jnp.bfloat16)
        lo = (x - hi.astype(jnp.float32)).astype(jnp.bfloat16)
        return jnp.concatenate([hi, lo], axis=0)

    ent2 = _hilo(ent)
    rel2 = _hilo(rel)

    grid = batch // BM
    out = pl.pallas_call(
        _body,
        grid=(grid,),
        in_specs=[
            pl.BlockSpec((BM, 1), lambda i: (i, 0)),
            pl.BlockSpec((BM, 1), lambda i: (i, 0)),
            pl.BlockSpec((BM, 1), lambda i: (i, 0)),
            pl.BlockSpec((2 * TBL, 2 * EMB), lambda i: (0, 0)),
            pl.BlockSpec((2 * TBL, 4 * EMB), lambda i: (0, 0)),
        ],
        out_specs=pl.BlockSpec((BM, 1), lambda i: (i, 0)),
        out_shape=jax.ShapeDtypeStruct((batch, 1), jnp.float32),
    )(h, t, r, ent2, rel2)
    return out[:, 0]


# speedup vs baseline: 2.2579x; 1.0010x over previous
import functools

import jax
import jax.numpy as jnp
from jax import lax
from jax.experimental import pallas as pl
from jax.experimental.pallas import tpu as pltpu

GUMBEL_BETA = 0.01
EULER_GAMMA = 0.5772156649015329
EMB = 128
TBL = 1024
BM = 1024


def _log1pexp(x):
    return jnp.log1p(jnp.exp(x))


def _logaddexp(a, b):
    m = jnp.maximum(a, b)
    return m + _log1pexp(-jnp.abs(a - b))


def _softplus(x):
    return jnp.maximum(x, 0.0) + _log1pexp(-jnp.abs(x))


def _log_volume(delta):
    eps = jnp.finfo(jnp.float32).tiny
    sp = _softplus(delta - 2.0 * EULER_GAMMA * GUMBEL_BETA)
    return jnp.sum(jnp.log(jnp.maximum(sp, eps)), axis=-1, keepdims=True)


def _body(h_ref, t_ref, r_ref, ent_ref, rel_ref, out_ref, gh_s, gt_s, gr_s):
    bf16 = jnp.bfloat16
    f32 = jnp.float32
    iota = lax.broadcasted_iota(jnp.int32, (BM, 2 * TBL), 1) & (TBL - 1)
    oh_h = (iota == h_ref[...]).astype(bf16)
    oh_t = (iota == t_ref[...]).astype(bf16)
    oh_r = (iota == r_ref[...]).astype(bf16)
    gh_s[...] = lax.dot(oh_h, ent_ref[...], preferred_element_type=f32)
    gt_s[...] = lax.dot(oh_t, ent_ref[...], preferred_element_type=f32)
    gr_s[...] = lax.dot(oh_r, rel_ref[...], preferred_element_type=f32)
    gh = gh_s[...]
    gt = gt_s[...]
    gr = gr_s[...]

    min_h = gh[:, :EMB]
    max_h = min_h + jnp.exp(gh[:, EMB:])
    delta_h = max_h - min_h
    trans_h = gr[:, 0:EMB]
    scale_h = jnp.maximum(gr[:, EMB:2 * EMB], 0.0)
    min_h = min_h + trans_h
    delta_h = delta_h * scale_h
    max_h = min_h + delta_h

    min_t = gt[:, :EMB]
    max_t = min_t + jnp.exp(gt[:, EMB:])
    delta_t = max_t - min_t
    trans_t = gr[:, 2 * EMB:3 * EMB]
    scale_t = jnp.maximum(gr[:, 3 * EMB:], 0.0)
    min_t = min_t + trans_t
    delta_t = delta_t * scale_t
    max_t = min_t + delta_t

    b = GUMBEL_BETA
    int_min = b * _logaddexp(min_h / b, min_t / b)
    int_min = jnp.maximum(int_min, jnp.maximum(min_h, min_t))
    int_max = -b * _logaddexp(-max_h / b, -max_t / b)
    int_max = jnp.minimum(int_max, jnp.minimum(max_h, max_t))

    li = _log_volume(int_max - int_min)
    lt = _log_volume(delta_t)
    out_ref[...] = jnp.exp(li - lt)


def kernel(ids, min_embedding, delta_embedding, rel_trans_for_head,
           rel_scale_for_head, rel_trans_for_tail, rel_scale_for_tail):
    batch = ids.shape[0]
    h = ids[:, 0:1]
    r = ids[:, 1:2]
    t = ids[:, 2:3]
    ent = jnp.concatenate([min_embedding[:TBL], delta_embedding[:TBL]], axis=1)
    pad = TBL - rel_trans_for_head.shape[0]
    rel = jnp.concatenate([
        jnp.pad(rel_trans_for_head, ((0, pad), (0, 0))),
        jnp.pad(rel_scale_for_head, ((0, pad), (0, 0))),
        jnp.pad(rel_trans_for_tail, ((0, pad), (0, 0))),
        jnp.pad(rel_scale_for_tail, ((0, pad), (0, 0))),
    ], axis=1)
    def _hilo(x):
        hi = x.astype(jnp.bfloat16)
        lo = (x - hi.astype(jnp.float32)).astype(jnp.bfloat16)
        return jnp.concatenate([hi, lo], axis=0)

    ent2 = _hilo(ent)
    rel2 = _hilo(rel)

    grid = batch // BM
    out = pl.pallas_call(
        _body,
        grid=(grid,),
        in_specs=[
            pl.BlockSpec((BM, 1), lambda i: (i, 0)),
            pl.BlockSpec((BM, 1), lambda i: (i, 0)),
            pl.BlockSpec((BM, 1), lambda i: (i, 0)),
            pl.BlockSpec((2 * TBL, 2 * EMB), lambda i: (0, 0)),
            pl.BlockSpec((2 * TBL, 4 * EMB), lambda i: (0, 0)),
        ],
        out_specs=pl.BlockSpec((BM, 1), lambda i: (i, 0)),
        out_shape=jax.ShapeDtypeStruct((batch, 1), jnp.float32),
        scratch_shapes=[
            pltpu.VMEM((BM, 2 * EMB), jnp.float32),
            pltpu.VMEM((BM, 2 * EMB), jnp.float32),
            pltpu.VMEM((BM, 4 * EMB), jnp.float32),
        ],
    )(h, t, r, ent2, rel2)
    return out[:, 0]


# speedup vs baseline: 2.2851x; 1.0121x over previous
import jax
import jax.numpy as jnp
from jax import lax
from jax.experimental import pallas as pl
from jax.experimental.pallas import tpu as pltpu

GUMBEL_BETA = 0.01
EULER_GAMMA = 0.5772156649015329
EMB = 128
TBL = 1024
BM = 1024


def _log1pexp(x):
    return jnp.log1p(jnp.exp(x))


def _logaddexp(a, b):
    m = jnp.maximum(a, b)
    return m + _log1pexp(-jnp.abs(a - b))


def _softplus(x):
    return jnp.maximum(x, 0.0) + _log1pexp(-jnp.abs(x))


def _log_volume(delta):
    eps = jnp.finfo(jnp.float32).tiny
    sp = _softplus(delta - 2.0 * EULER_GAMMA * GUMBEL_BETA)
    return jnp.sum(jnp.log(jnp.maximum(sp, eps)), axis=-1, keepdims=True)


def _body(h_ref, t_ref, r_ref, ent_ref, rel_ref, out_ref, ent2_s, rel2_s,
          gh_s, gt_s, gr_s):
    bf16 = jnp.bfloat16
    f32 = jnp.float32

    @pl.when(pl.program_id(0) == 0)
    def _fill_tables():
        for src, dst in ((ent_ref, ent2_s), (rel_ref, rel2_s)):
            x = src[...]
            hi = x.astype(bf16)
            lo = (x - hi.astype(f32)).astype(bf16)
            dst[:TBL, :] = hi
            dst[TBL:, :] = lo

    iota = lax.broadcasted_iota(jnp.int32, (BM, 2 * TBL), 1) & (TBL - 1)
    oh_h = (iota == h_ref[...]).astype(bf16)
    oh_t = (iota == t_ref[...]).astype(bf16)
    oh_r = (iota == r_ref[...]).astype(bf16)
    gh_s[...] = lax.dot(oh_h, ent2_s[...], preferred_element_type=f32)
    gt_s[...] = lax.dot(oh_t, ent2_s[...], preferred_element_type=f32)
    gr_s[...] = lax.dot(oh_r, rel2_s[...], preferred_element_type=f32)
    gh = gh_s[...]
    gt = gt_s[...]
    gr = gr_s[...]

    min_h = gh[:, :EMB]
    max_h = min_h + jnp.exp(gh[:, EMB:])
    delta_h = max_h - min_h
    trans_h = gr[:, 0:EMB]
    scale_h = jnp.maximum(gr[:, EMB:2 * EMB], 0.0)
    min_h = min_h + trans_h
    delta_h = delta_h * scale_h
    max_h = min_h + delta_h

    min_t = gt[:, :EMB]
    max_t = min_t + jnp.exp(gt[:, EMB:])
    delta_t = max_t - min_t
    trans_t = gr[:, 2 * EMB:3 * EMB]
    scale_t = jnp.maximum(gr[:, 3 * EMB:], 0.0)
    min_t = min_t + trans_t
    delta_t = delta_t * scale_t
    max_t = min_t + delta_t

    b = GUMBEL_BETA
    int_min = b * _logaddexp(min_h / b, min_t / b)
    int_min = jnp.maximum(int_min, jnp.maximum(min_h, min_t))
    int_max = -b * _logaddexp(-max_h / b, -max_t / b)
    int_max = jnp.minimum(int_max, jnp.minimum(max_h, max_t))

    li = _log_volume(int_max - int_min)
    lt = _log_volume(delta_t)
    out_ref[...] = jnp.exp(li - lt)


def kernel(ids, min_embedding, delta_embedding, rel_trans_for_head,
           rel_scale_for_head, rel_trans_for_tail, rel_scale_for_tail):
    batch = ids.shape[0]
    h = ids[:, 0:1]
    r = ids[:, 1:2]
    t = ids[:, 2:3]
    ent = jnp.concatenate([min_embedding[:TBL], delta_embedding[:TBL]], axis=1)
    pad = TBL - rel_trans_for_head.shape[0]
    rel = jnp.concatenate([
        jnp.pad(rel_trans_for_head, ((0, pad), (0, 0))),
        jnp.pad(rel_scale_for_head, ((0, pad), (0, 0))),
        jnp.pad(rel_trans_for_tail, ((0, pad), (0, 0))),
        jnp.pad(rel_scale_for_tail, ((0, pad), (0, 0))),
    ], axis=1)

    grid = batch // BM
    out = pl.pallas_call(
        _body,
        grid=(grid,),
        in_specs=[
            pl.BlockSpec((BM, 1), lambda i: (i, 0)),
            pl.BlockSpec((BM, 1), lambda i: (i, 0)),
            pl.BlockSpec((BM, 1), lambda i: (i, 0)),
            pl.BlockSpec((TBL, 2 * EMB), lambda i: (0, 0)),
            pl.BlockSpec((TBL, 4 * EMB), lambda i: (0, 0)),
        ],
        out_specs=pl.BlockSpec((BM, 1), lambda i: (i, 0)),
        out_shape=jax.ShapeDtypeStruct((batch, 1), jnp.float32),
        scratch_shapes=[
            pltpu.VMEM((2 * TBL, 2 * EMB), jnp.bfloat16),
            pltpu.VMEM((2 * TBL, 4 * EMB), jnp.bfloat16),
            pltpu.VMEM((BM, 2 * EMB), jnp.float32),
            pltpu.VMEM((BM, 2 * EMB), jnp.float32),
            pltpu.VMEM((BM, 4 * EMB), jnp.float32),
        ],
    )(h, t, r, ent, rel)
    return out[:, 0]


# speedup vs baseline: 2.3963x; 1.0487x over previous
import functools

import jax
import jax.numpy as jnp
from jax import lax
from jax.experimental import pallas as pl
from jax.experimental.pallas import tpu as pltpu
from jax.experimental.pallas import tpu_sc as plsc

GUMBEL_BETA = 0.01
EULER_GAMMA = 0.5772156649015329
EMB = 128
BM = 2048
CH = 64


def _sc_gather(h, t, r, emin, edel, relcat):
    info = plsc.get_sparse_core_info()
    nc, ns = info.num_cores, info.num_subcores
    nw = nc * ns
    batch = h.shape[0]
    bpw = batch // nw
    nch = bpw // CH
    f32 = jnp.float32
    mesh = plsc.VectorSubcoreMesh(core_axis_name="c", subcore_axis_name="s")

    @functools.partial(
        pl.kernel,
        out_type=[
            jax.ShapeDtypeStruct((batch, EMB), f32),
            jax.ShapeDtypeStruct((batch, EMB), f32),
            jax.ShapeDtypeStruct((batch, EMB), f32),
            jax.ShapeDtypeStruct((batch, EMB), f32),
            jax.ShapeDtypeStruct((batch, 4 * EMB), f32),
        ],
        mesh=mesh,
        scratch_types=[
            pltpu.VMEM((bpw,), jnp.int32),
            pltpu.VMEM((bpw,), jnp.int32),
            pltpu.VMEM((bpw,), jnp.int32),
            pltpu.VMEM((CH, EMB), f32),
            pltpu.VMEM((CH, EMB), f32),
            pltpu.VMEM((CH, EMB), f32),
            pltpu.VMEM((CH, EMB), f32),
            pltpu.VMEM((CH, 4 * EMB), f32),
            pltpu.SemaphoreType.DMA,
            pltpu.SemaphoreType.DMA,
        ],
    )
    def gather_kernel(h_hbm, t_hbm, r_hbm, emin_hbm, edel_hbm, rel_hbm,
                      o_mh, o_dh, o_mt, o_dt, o_gr,
                      idx_h, idx_t, idx_r, b_mh, b_dh, b_mt, b_dt, b_gr,
                      gsem, wsem):
        wid = lax.axis_index("s") * nc + lax.axis_index("c")
        base = wid * bpw
        pltpu.sync_copy(h_hbm.at[pl.ds(base, bpw)], idx_h)
        pltpu.sync_copy(t_hbm.at[pl.ds(base, bpw)], idx_t)
        pltpu.sync_copy(r_hbm.at[pl.ds(base, bpw)], idx_r)

        def chunk(c, carry):
            off = c * CH
            gs = [
                pltpu.async_copy(emin_hbm.at[idx_h.at[pl.ds(off, CH)]], b_mh, gsem),
                pltpu.async_copy(edel_hbm.at[idx_h.at[pl.ds(off, CH)]], b_dh, gsem),
                pltpu.async_copy(emin_hbm.at[idx_t.at[pl.ds(off, CH)]], b_mt, gsem),
                pltpu.async_copy(edel_hbm.at[idx_t.at[pl.ds(off, CH)]], b_dt, gsem),
                pltpu.async_copy(rel_hbm.at[idx_r.at[pl.ds(off, CH)]], b_gr, gsem),
            ]
            for g in gs:
                g.wait()
            ws = [
                pltpu.async_copy(b_mh, o_mh.at[pl.ds(base + off, CH)], wsem),
                pltpu.async_copy(b_dh, o_dh.at[pl.ds(base + off, CH)], wsem),
                pltpu.async_copy(b_mt, o_mt.at[pl.ds(base + off, CH)], wsem),
                pltpu.async_copy(b_dt, o_dt.at[pl.ds(base + off, CH)], wsem),
                pltpu.async_copy(b_gr, o_gr.at[pl.ds(base + off, CH)], wsem),
            ]
            for w in ws:
                w.wait()
            return carry

        lax.fori_loop(0, nch, chunk, 0)

    return gather_kernel(h, t, r, emin, edel, relcat)


def _log1pexp(x):
    return jnp.log1p(jnp.exp(x))


def _logaddexp(a, b):
    m = jnp.maximum(a, b)
    return m + _log1pexp(-jnp.abs(a - b))


def _softplus(x):
    return jnp.maximum(x, 0.0) + _log1pexp(-jnp.abs(x))


def _log_volume(delta):
    eps = jnp.finfo(jnp.float32).tiny
    sp = _softplus(delta - 2.0 * EULER_GAMMA * GUMBEL_BETA)
    return jnp.sum(jnp.log(jnp.maximum(sp, eps)), axis=-1, keepdims=True)


def _math_body(mh_ref, dh_ref, mt_ref, dt_ref, gr_ref, out_ref):
    gr = gr_ref[...]
    min_h = mh_ref[...]
    max_h = min_h + jnp.exp(dh_ref[...])
    delta_h = max_h - min_h
    trans_h = gr[:, 0:EMB]
    scale_h = jnp.maximum(gr[:, EMB:2 * EMB], 0.0)
    min_h = min_h + trans_h
    delta_h = delta_h * scale_h
    max_h = min_h + delta_h

    min_t = mt_ref[...]
    max_t = min_t + jnp.exp(dt_ref[...])
    delta_t = max_t - min_t
    trans_t = gr[:, 2 * EMB:3 * EMB]
    scale_t = jnp.maximum(gr[:, 3 * EMB:], 0.0)
    min_t = min_t + trans_t
    delta_t = delta_t * scale_t
    max_t = min_t + delta_t

    b = GUMBEL_BETA
    int_min = b * _logaddexp(min_h / b, min_t / b)
    int_min = jnp.maximum(int_min, jnp.maximum(min_h, min_t))
    int_max = -b * _logaddexp(-max_h / b, -max_t / b)
    int_max = jnp.minimum(int_max, jnp.minimum(max_h, max_t))

    li = _log_volume(int_max - int_min)
    lt = _log_volume(delta_t)
    out_ref[...] = jnp.exp(li - lt)


def kernel(ids, min_embedding, delta_embedding, rel_trans_for_head,
           rel_scale_for_head, rel_trans_for_tail, rel_scale_for_tail):
    batch = ids.shape[0]
    h = ids[:, 0]
    r = ids[:, 1]
    t = ids[:, 2]
    relcat = jnp.concatenate([rel_trans_for_head, rel_scale_for_head,
                              rel_trans_for_tail, rel_scale_for_tail], axis=1)

    mh, dh, mt, dt, gr = _sc_gather(h, t, r, min_embedding, delta_embedding,
                                    relcat)

    grid = batch // BM
    out = pl.pallas_call(
        _math_body,
        grid=(grid,),
        in_specs=[
            pl.BlockSpec((BM, EMB), lambda i: (i, 0)),
            pl.BlockSpec((BM, EMB), lambda i: (i, 0)),
            pl.BlockSpec((BM, EMB), lambda i: (i, 0)),
            pl.BlockSpec((BM, EMB), lambda i: (i, 0)),
            pl.BlockSpec((BM, 4 * EMB), lambda i: (i, 0)),
        ],
        out_specs=pl.BlockSpec((BM, 1), lambda i: (i, 0)),
        out_shape=jax.ShapeDtypeStruct((batch, 1), jnp.float32),
    )(mh, dh, mt, dt, gr)
    return out[:, 0]


# speedup vs baseline: 2.4215x; 1.0105x over previous
import functools

import jax
import jax.numpy as jnp
from jax import lax
from jax.experimental import pallas as pl
from jax.experimental.pallas import tpu as pltpu
from jax.experimental.pallas import tpu_sc as plsc

GUMBEL_BETA = 0.01
EULER_GAMMA = 0.5772156649015329
EMB = 128
BM = 2048
CH = 32


def _sc_gather(h, t, r, emin, edel, relcat):
    info = plsc.get_sparse_core_info()
    nc, ns = info.num_cores, info.num_subcores
    nw = nc * ns
    batch = h.shape[0]
    bpw = batch // nw
    nch = bpw // CH
    f32 = jnp.float32
    mesh = plsc.VectorSubcoreMesh(core_axis_name="c", subcore_axis_name="s")

    @functools.partial(
        pl.kernel,
        out_type=[
            jax.ShapeDtypeStruct((batch, EMB), f32),
            jax.ShapeDtypeStruct((batch, EMB), f32),
            jax.ShapeDtypeStruct((batch, EMB), f32),
            jax.ShapeDtypeStruct((batch, EMB), f32),
            jax.ShapeDtypeStruct((batch, 4 * EMB), f32),
        ],
        mesh=mesh,
        scratch_types=[
            pltpu.VMEM((bpw,), jnp.int32),
            pltpu.VMEM((bpw,), jnp.int32),
            pltpu.VMEM((bpw,), jnp.int32),
            pltpu.VMEM((2, CH, EMB), f32),
            pltpu.VMEM((2, CH, EMB), f32),
            pltpu.VMEM((2, CH, EMB), f32),
            pltpu.VMEM((2, CH, EMB), f32),
            pltpu.VMEM((2, CH, 4 * EMB), f32),
            pltpu.SemaphoreType.DMA,
            pltpu.SemaphoreType.DMA,
            pltpu.SemaphoreType.DMA,
            pltpu.SemaphoreType.DMA,
        ],
    )
    def gather_kernel(h_hbm, t_hbm, r_hbm, emin_hbm, edel_hbm, rel_hbm,
                      o_mh, o_dh, o_mt, o_dt, o_gr,
                      idx_h, idx_t, idx_r, b_mh, b_dh, b_mt, b_dt, b_gr,
                      gsem0, gsem1, wsem0, wsem1):
        wid = lax.axis_index("s") * nc + lax.axis_index("c")
        base = wid * bpw
        pltpu.sync_copy(h_hbm.at[pl.ds(base, bpw)], idx_h)
        pltpu.sync_copy(t_hbm.at[pl.ds(base, bpw)], idx_t)
        pltpu.sync_copy(r_hbm.at[pl.ds(base, bpw)], idx_r)

        gsems = (gsem0, gsem1)
        wsems = (wsem0, wsem1)
        bufs = (b_mh, b_dh, b_mt, b_dt, b_gr)
        outs = (o_mh, o_dh, o_mt, o_dt, o_gr)
        idxs = (idx_h, idx_h, idx_t, idx_t, idx_r)
        tbls = (emin_hbm, edel_hbm, emin_hbm, edel_hbm, rel_hbm)

        def fire_gathers(c, s):
            off = c * CH
            for tbl, idx, buf in zip(tbls, idxs, bufs):
                pltpu.async_copy(tbl.at[idx.at[pl.ds(off, CH)]], buf.at[s],
                                 gsems[s])

        def wait_gathers(s):
            for tbl, idx, buf in zip(tbls, idxs, bufs):
                pltpu.make_async_copy(tbl.at[idx.at[pl.ds(0, CH)]], buf.at[s],
                                      gsems[s]).wait()

        def fire_writes(c, s):
            off = c * CH
            for buf, out in zip(bufs, outs):
                pltpu.async_copy(buf.at[s], out.at[pl.ds(base + off, CH)],
                                 wsems[s])

        def wait_writes(s):
            for buf, out in zip(bufs, outs):
                pltpu.make_async_copy(buf.at[s], out.at[pl.ds(base, CH)],
                                      wsems[s]).wait()

        fire_gathers(0, 0)

        def pair(k, carry):
            c0 = 2 * k

            @pl.when(k > 0)
            def _():
                wait_writes(1)

            fire_gathers(c0 + 1, 1)
            wait_gathers(0)
            fire_writes(c0, 0)
            wait_gathers(1)
            fire_writes(c0 + 1, 1)

            @pl.when(k < nch // 2 - 1)
            def _():
                wait_writes(0)
                fire_gathers(c0 + 2, 0)

            return carry

        lax.fori_loop(0, nch // 2, pair, 0)
        wait_writes(0)
        wait_writes(1)

    return gather_kernel(h, t, r, emin, edel, relcat)


def _log1pexp(x):
    return jnp.log1p(jnp.exp(x))


def _logaddexp(a, b):
    m = jnp.maximum(a, b)
    return m + _log1pexp(-jnp.abs(a - b))


def _softplus(x):
    return jnp.maximum(x, 0.0) + _log1pexp(-jnp.abs(x))


def _log_volume(delta):
    eps = jnp.finfo(jnp.float32).tiny
    sp = _softplus(delta - 2.0 * EULER_GAMMA * GUMBEL_BETA)
    return jnp.sum(jnp.log(jnp.maximum(sp, eps)), axis=-1, keepdims=True)


def _math_body(mh_ref, dh_ref, mt_ref, dt_ref, gr_ref, out_ref):
    gr = gr_ref[...]
    min_h = mh_ref[...]
    max_h = min_h + jnp.exp(dh_ref[...])
    delta_h = max_h - min_h
    trans_h = gr[:, 0:EMB]
    scale_h = jnp.maximum(gr[:, EMB:2 * EMB], 0.0)
    min_h = min_h + trans_h
    delta_h = delta_h * scale_h
    max_h = min_h + delta_h

    min_t = mt_ref[...]
    max_t = min_t + jnp.exp(dt_ref[...])
    delta_t = max_t - min_t
    trans_t = gr[:, 2 * EMB:3 * EMB]
    scale_t = jnp.maximum(gr[:, 3 * EMB:], 0.0)
    min_t = min_t + trans_t
    delta_t = delta_t * scale_t
    max_t = min_t + delta_t

    b = GUMBEL_BETA
    int_min = b * _logaddexp(min_h / b, min_t / b)
    int_min = jnp.maximum(int_min, jnp.maximum(min_h, min_t))
    int_max = -b * _logaddexp(-max_h / b, -max_t / b)
    int_max = jnp.minimum(int_max, jnp.minimum(max_h, max_t))

    li = _log_volume(int_max - int_min)
    lt = _log_volume(delta_t)
    out_ref[...] = jnp.exp(li - lt)


def kernel(ids, min_embedding, delta_embedding, rel_trans_for_head,
           rel_scale_for_head, rel_trans_for_tail, rel_scale_for_tail):
    batch = ids.shape[0]
    h = ids[:, 0]
    r = ids[:, 1]
    t = ids[:, 2]
    relcat = jnp.concatenate([rel_trans_for_head, rel_scale_for_head,
                              rel_trans_for_tail, rel_scale_for_tail], axis=1)

    mh, dh, mt, dt, gr = _sc_gather(h, t, r, min_embedding, delta_embedding,
                                    relcat)

    grid = batch // BM
    out = pl.pallas_call(
        _math_body,
        grid=(grid,),
        in_specs=[
            pl.BlockSpec((BM, EMB), lambda i: (i, 0)),
            pl.BlockSpec((BM, EMB), lambda i: (i, 0)),
            pl.BlockSpec((BM, EMB), lambda i: (i, 0)),
            pl.BlockSpec((BM, EMB), lambda i: (i, 0)),
            pl.BlockSpec((BM, 4 * EMB), lambda i: (i, 0)),
        ],
        out_specs=pl.BlockSpec((BM, 1), lambda i: (i, 0)),
        out_shape=jax.ShapeDtypeStruct((batch, 1), jnp.float32),
    )(mh, dh, mt, dt, gr)
    return out[:, 0]
